# Initial kernel scaffold; baseline (speedup 1.0000x reference)
import jax
import jax.numpy as jnp
from jax.experimental import pallas as pl


def kernel(x, edge_index, W0, b0, g0, be0, W1, b1, g1, be1, W2, b2, g2, be2, lin1_W, lin1_b, lin2_W, lin2_b):
    raise NotImplementedError("write your pallas kernel here")


# speedup vs baseline: 12.9914x; 12.9914x over previous
import dataclasses
import functools

import jax
import jax.numpy as jnp
from jax import lax
from jax.experimental import pallas as pl
from jax.experimental.pallas import tpu as pltpu
from jax.experimental.pallas import tpu_sc as plsc

N = 10000
E = 320000
F_IN = 128
H = 64

NC = 2
NS = 16
NW = NC * NS
LANES = 16

EPW = E // NW
K = 80
NCHUNK = EPW // K
NRC = N // K
RCPT = (NRC + NS - 1) // NS

_mesh = plsc.VectorSubcoreMesh(core_axis_name="c", subcore_axis_name="s")

_sc_params = pltpu.CompilerParams()
if "needs_layout_passes" in pltpu.CompilerParams.__dataclass_fields__:
    _sc_params = dataclasses.replace(_sc_params, needs_layout_passes=False)
if "use_tc_tiling_on_sc" in pltpu.CompilerParams.__dataclass_fields__:
    _sc_params = dataclasses.replace(_sc_params, use_tc_tiling_on_sc=False)


@functools.partial(
    pl.kernel,
    out_type=jax.ShapeDtypeStruct((NW, N), jnp.float32),
    mesh=_mesh,
    scratch_types=[
        pltpu.VMEM((N,), jnp.float32),
        pltpu.VMEM((K,), jnp.int32),
    ],
    compiler_params=_sc_params,
)
def _sc_degree(dst_hbm, out_hbm, hist, idxv):
    c = lax.axis_index("c")
    s = lax.axis_index("s")
    wid = c * NS + s

    @pl.loop(0, N // LANES)
    def _(i):
        hist[pl.ds(i * LANES, LANES)] = jnp.zeros((LANES,), jnp.float32)

    ebase = wid * EPW

    @pl.loop(0, NCHUNK)
    def _(ci):
        pltpu.sync_copy(dst_hbm.at[pl.ds(ebase + ci * K, K)], idxv)

        @pl.loop(0, K // LANES)
        def _(j):
            d = idxv[pl.ds(j * LANES, LANES)]
            plsc.addupdate_scatter(hist, [d], jnp.ones((LANES,), jnp.float32))

    pltpu.sync_copy(hist, out_hbm.at[wid])


@functools.partial(
    pl.kernel,
    out_type=jax.ShapeDtypeStruct((NC, N, H), jnp.float32),
    mesh=_mesh,
    scratch_types=[
        pltpu.VMEM((K,), jnp.int32),
        pltpu.VMEM((K,), jnp.int32),
        pltpu.VMEM((K, H), jnp.float32),
        pltpu.VMEM_SHARED((N, H), jnp.float32),
        pltpu.SemaphoreType.DMA,
    ],
    compiler_params=_sc_params,
)
def _sc_aggregate(ht_hbm, src_hbm, dst_hbm, out_hbm, isrc, idst, rows, acc, sem):
    c = lax.axis_index("c")
    s = lax.axis_index("s")
    wid = c * NS + s

    @pl.loop(0, K)
    def _(i):
        @pl.loop(0, H // LANES)
        def _(j):
            rows[i, pl.ds(j * LANES, LANES)] = jnp.zeros((LANES,), jnp.float32)

    @pl.loop(0, RCPT)
    def _(t):
        q = s + NS * t

        @pl.when(q < NRC)
        def _():
            pltpu.sync_copy(rows, acc.at[pl.ds(q * K, K)])

    plsc.subcore_barrier()

    ebase = wid * EPW

    @pl.loop(0, NCHUNK)
    def _(ci):
        off = ebase + ci * K
        pltpu.sync_copy(src_hbm.at[pl.ds(off, K)], isrc)
        pltpu.sync_copy(dst_hbm.at[pl.ds(off, K)], idst)
        pltpu.async_copy(ht_hbm.at[isrc], rows, sem).wait()
        pltpu.sync_copy(rows, acc.at[idst], add=True)

    plsc.subcore_barrier()

    @pl.loop(0, RCPT)
    def _(t):
        q = s + NS * t

        @pl.when(q < NRC)
        def _():
            pltpu.sync_copy(acc.at[pl.ds(q * K, K)], out_hbm.at[c, pl.ds(q * K, K)])


def _tc_prep_body(hist_ref, x_ref, w0_ref, dinv_ref, ht0_ref):
    deg = 1.0 + jnp.sum(hist_ref[...], axis=0)
    dinv = lax.rsqrt(deg)[:, None]
    dinv_ref[...] = dinv
    h0 = jnp.dot(x_ref[...], w0_ref[...], preferred_element_type=jnp.float32)
    ht0_ref[...] = dinv * h0


def _tc_prep(hist, x, W0):
    return pl.pallas_call(
        _tc_prep_body,
        out_shape=(
            jax.ShapeDtypeStruct((N, 1), jnp.float32),
            jax.ShapeDtypeStruct((N, H), jnp.float32),
        ),
    )(hist, x, W0)


def _bn_relu(out, g, be, eps=1e-5):
    mu = jnp.mean(out, axis=0, keepdims=True)
    xc = out - mu
    var = jnp.mean(xc * xc, axis=0, keepdims=True)
    return jnp.maximum(xc * lax.rsqrt(var + eps) * g + be, 0.0)


def _tc_mid_body(p_ref, ht_ref, dinv_ref, b_ref, g_ref, be_ref, wn_ref, htn_ref):
    ht = ht_ref[...]
    dinv = dinv_ref[...]
    s = p_ref[0] + p_ref[1] + ht
    out = dinv * s + b_ref[...]
    y = _bn_relu(out, g_ref[...], be_ref[...])
    htn_ref[...] = dinv * jnp.dot(y, wn_ref[...], preferred_element_type=jnp.float32)


def _tc_mid(p, ht, dinv, b, g, be, Wn):
    return pl.pallas_call(
        _tc_mid_body,
        out_shape=jax.ShapeDtypeStruct((N, H), jnp.float32),
    )(p, ht, dinv, b.reshape(1, H), g.reshape(1, H), be.reshape(1, H), Wn)


def _tc_final_body(p_ref, ht_ref, dinv_ref, b_ref, g_ref, be_ref,
                   w1_ref, b1_ref, w2_ref, b2_ref, o_ref):
    ht = ht_ref[...]
    dinv = dinv_ref[...]
    s = p_ref[0] + p_ref[1] + ht
    out = dinv * s + b_ref[...]
    y = _bn_relu(out, g_ref[...], be_ref[...])
    z = jnp.maximum(
        jnp.dot(y, w1_ref[...], preferred_element_type=jnp.float32) + b1_ref[...], 0.0)
    o_ref[...] = jnp.dot(z, w2_ref[...], preferred_element_type=jnp.float32) + b2_ref[...]


def _tc_final(p, ht, dinv, b, g, be, lin1_W, lin1_b, lin2_W, lin2_b):
    return pl.pallas_call(
        _tc_final_body,
        out_shape=jax.ShapeDtypeStruct((N, 1), jnp.float32),
    )(p, ht, dinv, b.reshape(1, H), g.reshape(1, H), be.reshape(1, H),
      lin1_W, lin1_b.reshape(1, H // 2), lin2_W, lin2_b.reshape(1, 1))


def kernel(x, edge_index, W0, b0, g0, be0, W1, b1, g1, be1, W2, b2, g2, be2,
           lin1_W, lin1_b, lin2_W, lin2_b):
    src = edge_index[0]
    dst = edge_index[1]

    hist = _sc_degree(dst)
    dinv, ht0 = _tc_prep(hist, x, W0)

    p0 = _sc_aggregate(ht0, src, dst)
    ht1 = _tc_mid(p0, ht0, dinv, b0, g0, be0, W1)

    p1 = _sc_aggregate(ht1, src, dst)
    ht2 = _tc_mid(p1, ht1, dinv, b1, g1, be1, W2)

    p2 = _sc_aggregate(ht2, src, dst)
    y = _tc_final(p2, ht2, dinv, b2, g2, be2, lin1_W, lin1_b, lin2_W, lin2_b)
    return y.reshape(N)
